# baseline (speedup 1.0000x reference)
import functools

import jax
import jax.numpy as jnp
from jax import lax
from jax.experimental import pallas as pl
from jax.experimental.pallas import tpu as pltpu
from jax.experimental.pallas import tpu_sc as plsc

NC = 2
NS = 16
NW = NC * NS
L = 16
R = 8
NBI = 2
NQ = 8
NBO = 8

assert NBI * NQ % NBO == 0


@functools.partial(jax.jit, static_argnums=(2, 3))
def _permute_rows(x2, perm, n_rows, d):
    rows_per_w = n_rows // NW
    n_chunks = rows_per_w // R
    dq = d // NQ
    mesh = plsc.VectorSubcoreMesh(core_axis_name="c", subcore_axis_name="s")

    def body(x_hbm, perm_hbm, out_hbm, perm_v, *bufs):
        ins = bufs[:NBI]
        oqs = bufs[NBI:NBI + NBO]
        isems = bufs[NBI + NBO:NBI + NBO + NBI]
        osems = bufs[NBI + NBO + NBI:]
        wid = lax.axis_index("s") * NC + lax.axis_index("c")
        base_r = wid * rows_per_w

        def start_in(c, b):
            pltpu.async_copy(
                x_hbm.at[pl.ds(base_r + c * R, R), :], ins[b], isems[b])

        def wait_in(b):
            pltpu.make_async_copy(
                x_hbm.at[pl.ds(0, R), :], ins[b], isems[b]).wait()

        def start_out(c, q, s):
            pltpu.async_copy(
                oqs[s],
                out_hbm.at[pl.ds(base_r + c * R, R), pl.ds(q * dq, dq)],
                osems[s])

        def wait_out(s):
            pltpu.make_async_copy(
                oqs[s], out_hbm.at[pl.ds(0, R), pl.ds(0, dq)], osems[s]).wait()

        def gather_chunk(c, b, first_round):
            iv = ins[b]
            wait_in(b)
            for q in range(NQ):
                s = (b * NQ + q) % NBO
                if not (first_round and b * NQ + q < NBO):
                    wait_out(s)
                ov = oqs[s]

                @plsc.parallel_loop(q * (dq // L), (q + 1) * (dq // L),
                                    unroll=2)
                def jloop(j):
                    idx = perm_v[pl.ds(j * L, L)]
                    jq = j - q * (dq // L)
                    for r in range(R):
                        rvec = jnp.full((L,), r, dtype=jnp.int32)
                        ov[r, pl.ds(jq * L, L)] = plsc.load_gather(
                            iv, [rvec, idx])

                start_out(c, q, s)

        for b in range(NBI):
            start_in(b, b)
        pltpu.sync_copy(perm_hbm, perm_v)
        for b in range(NBI):
            gather_chunk(b, b, True)
            start_in(b + NBI, b)

        n_main_groups = (n_chunks - 2 * NBI) // NBI

        @pl.loop(NBI, NBI + n_main_groups * NBI, step=NBI)
        def main(g):
            for b in range(NBI):
                gather_chunk(g + b, b, False)
                start_in(g + b + NBI, b)

        for c in range(NBI + n_main_groups * NBI, n_chunks):
            gather_chunk(c, c % NBI, False)
            if c + NBI < n_chunks:
                start_in(c + NBI, c % NBI)
        for s in range(NBO):
            wait_out(s)

    fn = pl.kernel(
        body,
        out_type=jax.ShapeDtypeStruct((n_rows, d), jnp.float32),
        mesh=mesh,
        scratch_types=(
            [pltpu.VMEM((d,), jnp.int32)]
            + [pltpu.VMEM((R, d), jnp.float32)] * NBI
            + [pltpu.VMEM((R, d // NQ), jnp.float32)] * NBO
            + [pltpu.SemaphoreType.DMA] * (NBI + NBO)
        ),
        compiler_params=pltpu.CompilerParams(needs_layout_passes=False),
    )
    return fn(x2, perm)


def kernel(x, perm):
    b, s, d = x.shape
    x2 = x.reshape(b * s, d)
    out = _permute_rows(x2, perm, b * s, d)
    return out.reshape(b, s, d)


# speedup vs baseline: 1.0955x; 1.0955x over previous
import functools

import jax
import jax.numpy as jnp
from jax import lax
from jax.experimental import pallas as pl
from jax.experimental.pallas import tpu as pltpu
from jax.experimental.pallas import tpu_sc as plsc

NC = 2
NS = 16
NW = NC * NS
L = 16
R = 8
NBI = 2
NQ = 2
NBO = 2

assert NBI * NQ % NBO == 0


@functools.partial(jax.jit, static_argnums=(2, 3))
def _permute_rows(x2, perm, n_rows, d):
    rows_per_w = n_rows // NW
    n_chunks = rows_per_w // R
    dq = d // NQ
    mesh = plsc.VectorSubcoreMesh(core_axis_name="c", subcore_axis_name="s")

    def body(x_hbm, perm_hbm, out_hbm, perm_v, *bufs):
        ins = bufs[:NBI]
        oqs = bufs[NBI:NBI + NBO]
        isems = bufs[NBI + NBO:NBI + NBO + NBI]
        osems = bufs[NBI + NBO + NBI:]
        wid = lax.axis_index("s") * NC + lax.axis_index("c")
        base_r = wid * rows_per_w

        def start_in(c, b):
            pltpu.async_copy(
                x_hbm.at[pl.ds(base_r + c * R, R), :], ins[b], isems[b])

        def wait_in(b):
            pltpu.make_async_copy(
                x_hbm.at[pl.ds(0, R), :], ins[b], isems[b]).wait()

        def start_out(c, q, s):
            pltpu.async_copy(
                oqs[s],
                out_hbm.at[pl.ds(base_r + c * R, R), pl.ds(q * dq, dq)],
                osems[s])

        def wait_out(s):
            pltpu.make_async_copy(
                oqs[s], out_hbm.at[pl.ds(0, R), pl.ds(0, dq)], osems[s]).wait()

        def gather_chunk(c, b, first_round):
            iv = ins[b]
            wait_in(b)
            for q in range(NQ):
                s = (b * NQ + q) % NBO
                if not (first_round and b * NQ + q < NBO):
                    wait_out(s)
                ov = oqs[s]

                @plsc.parallel_loop(q * (dq // L), (q + 1) * (dq // L),
                                    unroll=2)
                def jloop(j):
                    idx = perm_v[pl.ds(j * L, L)]
                    jq = j - q * (dq // L)
                    for r in range(R):
                        rvec = jnp.full((L,), r, dtype=jnp.int32)
                        ov[r, pl.ds(jq * L, L)] = plsc.load_gather(
                            iv, [rvec, idx])

                start_out(c, q, s)

        for b in range(NBI):
            start_in(b, b)
        pltpu.sync_copy(perm_hbm, perm_v)
        for b in range(NBI):
            gather_chunk(b, b, True)
            start_in(b + NBI, b)

        n_main_groups = (n_chunks - 2 * NBI) // NBI

        @pl.loop(NBI, NBI + n_main_groups * NBI, step=NBI)
        def main(g):
            for b in range(NBI):
                gather_chunk(g + b, b, False)
                start_in(g + b + NBI, b)

        for c in range(NBI + n_main_groups * NBI, n_chunks):
            gather_chunk(c, c % NBI, False)
            if c + NBI < n_chunks:
                start_in(c + NBI, c % NBI)
        for s in range(NBO):
            wait_out(s)

    fn = pl.kernel(
        body,
        out_type=jax.ShapeDtypeStruct((n_rows, d), jnp.float32),
        mesh=mesh,
        scratch_types=(
            [pltpu.VMEM((d,), jnp.int32)]
            + [pltpu.VMEM((R, d), jnp.float32)] * NBI
            + [pltpu.VMEM((R, d // NQ), jnp.float32)] * NBO
            + [pltpu.SemaphoreType.DMA] * (NBI + NBO)
        ),
        compiler_params=pltpu.CompilerParams(needs_layout_passes=False),
    )
    return fn(x2, perm)


def kernel(x, perm):
    b, s, d = x.shape
    x2 = x.reshape(b * s, d)
    out = _permute_rows(x2, perm, b * s, d)
    return out.reshape(b, s, d)
